# Initial kernel scaffold; baseline (speedup 1.0000x reference)
import jax
import jax.numpy as jnp
from jax.experimental import pallas as pl


def kernel(x, W1, b1, W2, b2):
    raise NotImplementedError("write your pallas kernel here")


# speedup vs baseline: 1.0848x; 1.0848x over previous
import functools

import jax
import jax.numpy as jnp
from jax.experimental import pallas as pl
from jax.experimental.pallas import tpu as pltpu


def _gating_body(x_ref, w1_ref, b1_ref, w2_ref, b2_ref,
                 wts_ref, idx_ref, ent_ref,
                 acc_ref, ent_acc_ref, *, num_h, k, num_experts):
    m = pl.program_id(0)
    h = pl.program_id(1)

    hidden = jnp.dot(x_ref[...], w1_ref[...],
                     preferred_element_type=jnp.float32)
    hidden = jnp.maximum(hidden + b1_ref[...], 0.0)
    part = jnp.dot(hidden, w2_ref[...], preferred_element_type=jnp.float32)

    @pl.when(h == 0)
    def _():
        acc_ref[...] = part

    @pl.when(h != 0)
    def _():
        acc_ref[...] += part

    @pl.when(jnp.logical_and(m == 0, h == 0))
    def _():
        ent_acc_ref[0] = 0.0

    @pl.when(h == num_h - 1)
    def _():
        logits = acc_ref[...] + b2_ref[...]
        mx = jnp.max(logits, axis=1, keepdims=True)
        e = jnp.exp(logits - mx)
        s = jnp.sum(e, axis=1, keepdims=True)
        probs = e / s

        ent = -jnp.sum(probs * jnp.log(probs + 1e-10), axis=1)
        ent_acc_ref[0] += jnp.sum(ent)

        bm = probs.shape[0]
        lane = jax.lax.broadcasted_iota(jnp.int32, (bm, num_experts), 1)
        work = probs
        vals = []
        idxs = []
        for _ in range(k):
            mj = jnp.max(work, axis=1, keepdims=True)
            aj = jnp.min(jnp.where(work == mj, lane, num_experts), axis=1,
                         keepdims=True)
            vals.append(mj)
            idxs.append(aj)
            work = jnp.where(lane == aj, -1.0, work)
        w = jnp.concatenate(vals, axis=1)
        wts_ref[...] = w / jnp.sum(w, axis=1, keepdims=True)
        idx_ref[...] = jnp.concatenate(idxs, axis=1)

        @pl.when(m == pl.num_programs(0) - 1)
        def _():
            ent_ref[0] = ent_acc_ref[0]


def kernel(x, W1, b1, W2, b2):
    tokens, in_dim = x.shape
    hidden_dim, num_experts = W2.shape
    k = 8
    bm = 512
    bh = 512
    num_m = tokens // bm
    num_h = hidden_dim // bh

    b1r = b1.reshape(1, hidden_dim)
    b2r = b2.reshape(1, num_experts)

    body = functools.partial(_gating_body, num_h=num_h, k=k,
                             num_experts=num_experts)

    wts, idx, ent_sum = pl.pallas_call(
        body,
        grid=(num_m, num_h),
        in_specs=[
            pl.BlockSpec((bm, in_dim), lambda m, h: (m, 0)),
            pl.BlockSpec((in_dim, bh), lambda m, h: (0, h)),
            pl.BlockSpec((1, bh), lambda m, h: (0, h)),
            pl.BlockSpec((bh, num_experts), lambda m, h: (h, 0)),
            pl.BlockSpec((1, num_experts), lambda m, h: (0, 0)),
        ],
        out_specs=[
            pl.BlockSpec((bm, k), lambda m, h: (m, 0)),
            pl.BlockSpec((bm, k), lambda m, h: (m, 0)),
            pl.BlockSpec(memory_space=pltpu.SMEM),
        ],
        out_shape=[
            jax.ShapeDtypeStruct((tokens, k), jnp.float32),
            jax.ShapeDtypeStruct((tokens, k), jnp.int32),
            jax.ShapeDtypeStruct((1,), jnp.float32),
        ],
        scratch_shapes=[
            pltpu.VMEM((bm, num_experts), jnp.float32),
            pltpu.SMEM((1,), jnp.float32),
        ],
    )(x, W1, b1r, W2, b2r)

    uncertainty = (ent_sum[0] / tokens) / jnp.log(jnp.float32(num_experts))
    return wts, idx, uncertainty


# speedup vs baseline: 1.1071x; 1.0205x over previous
import functools

import jax
import jax.numpy as jnp
from jax.experimental import pallas as pl
from jax.experimental.pallas import tpu as pltpu


def _gating_body(x_ref, w1_ref, b1_ref, w2_ref, b2_ref,
                 wts_ref, idx_ref, ent_ref,
                 acc_ref, ent_acc_ref, *, num_h, k, num_experts):
    m = pl.program_id(0)
    h = pl.program_id(1)

    hidden = jnp.dot(x_ref[...], w1_ref[...],
                     preferred_element_type=jnp.float32)
    hidden = jnp.maximum(hidden + b1_ref[...], 0.0)
    part = jnp.dot(hidden, w2_ref[...], preferred_element_type=jnp.float32)

    @pl.when(h == 0)
    def _():
        acc_ref[...] = part

    @pl.when(h != 0)
    def _():
        acc_ref[...] += part

    @pl.when(jnp.logical_and(m == 0, h == 0))
    def _():
        ent_acc_ref[0] = 0.0

    @pl.when(h == num_h - 1)
    def _():
        logits = acc_ref[...] + b2_ref[...]
        mx = jnp.max(logits, axis=1, keepdims=True)
        e = jnp.exp(logits - mx)
        s = jnp.sum(e, axis=1, keepdims=True)
        probs = e / s

        ent = -jnp.sum(probs * jnp.log(probs + 1e-10), axis=1)
        ent_acc_ref[0] += jnp.sum(ent)

        bm = probs.shape[0]
        lane = jax.lax.broadcasted_iota(jnp.int32, (bm, num_experts), 1)
        pbits = jax.lax.bitcast_convert_type(probs, jnp.int32)
        key = jnp.bitwise_or(jnp.bitwise_and(pbits, jnp.int32(~63)),
                             (num_experts - 1) - lane)
        kms = []
        for _ in range(k):
            km = jnp.max(key, axis=1, keepdims=True)
            kms.append(km)
            key = jnp.where(key == km, jnp.int32(-1), key)
        km8 = jnp.concatenate(kms, axis=1)
        idx_ref[...] = (num_experts - 1) - jnp.bitwise_and(km8, jnp.int32(63))
        w = jax.lax.bitcast_convert_type(
            jnp.bitwise_and(km8, jnp.int32(~63)), jnp.float32)
        wts_ref[...] = w / jnp.sum(w, axis=1, keepdims=True)

        @pl.when(m == pl.num_programs(0) - 1)
        def _():
            ent_ref[0] = ent_acc_ref[0]


def kernel(x, W1, b1, W2, b2):
    tokens, in_dim = x.shape
    hidden_dim, num_experts = W2.shape
    k = 8
    bm = 512
    bh = 512
    num_m = tokens // bm
    num_h = hidden_dim // bh

    b1r = b1.reshape(1, hidden_dim)
    b2r = b2.reshape(1, num_experts)

    body = functools.partial(_gating_body, num_h=num_h, k=k,
                             num_experts=num_experts)

    wts, idx, ent_sum = pl.pallas_call(
        body,
        grid=(num_m, num_h),
        in_specs=[
            pl.BlockSpec((bm, in_dim), lambda m, h: (m, 0)),
            pl.BlockSpec((in_dim, bh), lambda m, h: (0, h)),
            pl.BlockSpec((1, bh), lambda m, h: (0, h)),
            pl.BlockSpec((bh, num_experts), lambda m, h: (h, 0)),
            pl.BlockSpec((1, num_experts), lambda m, h: (0, 0)),
        ],
        out_specs=[
            pl.BlockSpec((bm, k), lambda m, h: (m, 0)),
            pl.BlockSpec((bm, k), lambda m, h: (m, 0)),
            pl.BlockSpec(memory_space=pltpu.SMEM),
        ],
        out_shape=[
            jax.ShapeDtypeStruct((tokens, k), jnp.float32),
            jax.ShapeDtypeStruct((tokens, k), jnp.int32),
            jax.ShapeDtypeStruct((1,), jnp.float32),
        ],
        scratch_shapes=[
            pltpu.VMEM((bm, num_experts), jnp.float32),
            pltpu.SMEM((1,), jnp.float32),
        ],
    )(x, W1, b1r, W2, b2r)

    uncertainty = (ent_sum[0] / tokens) / jnp.log(jnp.float32(num_experts))
    return wts, idx, uncertainty


# speedup vs baseline: 1.3753x; 1.2423x over previous
import functools

import jax
import jax.numpy as jnp
from jax.experimental import pallas as pl
from jax.experimental.pallas import tpu as pltpu


def _gating_body(x_ref, w1_ref, b1_ref, w2_ref, b2_ref,
                 wts_ref, idx_ref, ent_ref,
                 acc_ref, ent_acc_ref, *, num_h, k, num_experts):
    m = pl.program_id(0)
    h = pl.program_id(1)

    hidden = jnp.dot(x_ref[...], w1_ref[...],
                     preferred_element_type=jnp.float32)
    hidden = jnp.maximum(hidden + b1_ref[...], 0.0)
    part = jnp.dot(hidden, w2_ref[...], preferred_element_type=jnp.float32)

    @pl.when(h == 0)
    def _():
        acc_ref[...] = part

    @pl.when(h != 0)
    def _():
        acc_ref[...] += part

    @pl.when(jnp.logical_and(m == 0, h == 0))
    def _():
        ent_acc_ref[0] = 0.0

    @pl.when(h == num_h - 1)
    def _():
        logits = acc_ref[...] + b2_ref[...]
        mx = jnp.max(logits, axis=1, keepdims=True)
        e = jnp.exp(logits - mx)
        s = jnp.sum(e, axis=1, keepdims=True)
        probs = e / s

        ent = -jnp.sum(probs * jnp.log(probs + 1e-10), axis=1)
        ent_acc_ref[0] += jnp.sum(ent)

        bm = probs.shape[0]
        lane = jax.lax.broadcasted_iota(jnp.int32, (bm, num_experts), 1)
        pbits = jax.lax.bitcast_convert_type(probs, jnp.int32)
        key = jnp.bitwise_or(jnp.bitwise_and(pbits, jnp.int32(~63)),
                             (num_experts - 1) - lane)
        kms = []
        for _ in range(k):
            km = jnp.max(key, axis=1, keepdims=True)
            kms.append(km)
            key = jnp.where(key == km, jnp.int32(-1), key)
        km8 = jnp.concatenate(kms, axis=1)
        idx_ref[...] = (num_experts - 1) - jnp.bitwise_and(km8, jnp.int32(63))
        w = jax.lax.bitcast_convert_type(
            jnp.bitwise_and(km8, jnp.int32(~63)), jnp.float32)
        wts_ref[...] = w / jnp.sum(w, axis=1, keepdims=True)

        @pl.when(m == pl.num_programs(0) - 1)
        def _():
            ent_ref[0] = ent_acc_ref[0]


def kernel(x, W1, b1, W2, b2):
    tokens, in_dim = x.shape
    hidden_dim, num_experts = W2.shape
    k = 8
    bm = 1024
    bh = 512
    num_m = tokens // bm
    num_h = hidden_dim // bh

    b1r = b1.reshape(1, hidden_dim)
    b2r = b2.reshape(1, num_experts)

    body = functools.partial(_gating_body, num_h=num_h, k=k,
                             num_experts=num_experts)

    wts, idx, ent_sum = pl.pallas_call(
        body,
        grid=(num_m, num_h),
        in_specs=[
            pl.BlockSpec((bm, in_dim), lambda m, h: (m, 0)),
            pl.BlockSpec((in_dim, bh), lambda m, h: (0, h)),
            pl.BlockSpec((1, bh), lambda m, h: (0, h)),
            pl.BlockSpec((bh, num_experts), lambda m, h: (h, 0)),
            pl.BlockSpec((1, num_experts), lambda m, h: (0, 0)),
        ],
        out_specs=[
            pl.BlockSpec((bm, k), lambda m, h: (m, 0)),
            pl.BlockSpec((bm, k), lambda m, h: (m, 0)),
            pl.BlockSpec(memory_space=pltpu.SMEM),
        ],
        out_shape=[
            jax.ShapeDtypeStruct((tokens, k), jnp.float32),
            jax.ShapeDtypeStruct((tokens, k), jnp.int32),
            jax.ShapeDtypeStruct((1,), jnp.float32),
        ],
        scratch_shapes=[
            pltpu.VMEM((bm, num_experts), jnp.float32),
            pltpu.SMEM((1,), jnp.float32),
        ],
    )(x, W1, b1r, W2, b2r)

    uncertainty = (ent_sum[0] / tokens) / jnp.log(jnp.float32(num_experts))
    return wts, idx, uncertainty


# speedup vs baseline: 1.3898x; 1.0105x over previous
import functools

import jax
import jax.numpy as jnp
from jax.experimental import pallas as pl
from jax.experimental.pallas import tpu as pltpu


def _gating_body(x_ref, w1_ref, b1_ref, w2_ref, b2_ref,
                 wts_ref, idx_ref, ent_ref,
                 acc_ref, ent_acc_ref, *, num_h, k, num_experts):
    m = pl.program_id(0)
    h = pl.program_id(1)

    hidden = jnp.dot(x_ref[...], w1_ref[...],
                     preferred_element_type=jnp.float32)
    hidden = jnp.maximum(hidden + b1_ref[...], 0.0)
    part = jnp.dot(hidden, w2_ref[...], preferred_element_type=jnp.float32)

    @pl.when(h == 0)
    def _():
        acc_ref[...] = part

    @pl.when(h != 0)
    def _():
        acc_ref[...] += part

    @pl.when(jnp.logical_and(m == 0, h == 0))
    def _():
        ent_acc_ref[0] = 0.0

    @pl.when(h == num_h - 1)
    def _():
        logits = acc_ref[...] + b2_ref[...]
        mx = jnp.max(logits, axis=1, keepdims=True)
        e = jnp.exp(logits - mx)
        s = jnp.sum(e, axis=1, keepdims=True)
        probs = e / s

        ent = -jnp.sum(probs * jnp.log(probs + 1e-10), axis=1)
        ent_acc_ref[0] += jnp.sum(ent)

        bm = probs.shape[0]
        lane = jax.lax.broadcasted_iota(jnp.int32, (bm, num_experts), 1)
        pbits = jax.lax.bitcast_convert_type(probs, jnp.int32)
        key = jnp.bitwise_or(jnp.bitwise_and(pbits, jnp.int32(~63)),
                             (num_experts - 1) - lane)
        kms = []
        for _ in range(k):
            km = jnp.max(key, axis=1, keepdims=True)
            kms.append(km)
            key = jnp.where(key == km, jnp.int32(-1), key)
        km8 = jnp.concatenate(kms, axis=1)
        idx_ref[...] = (num_experts - 1) - jnp.bitwise_and(km8, jnp.int32(63))
        w = jax.lax.bitcast_convert_type(
            jnp.bitwise_and(km8, jnp.int32(~63)), jnp.float32)
        wts_ref[...] = w / jnp.sum(w, axis=1, keepdims=True)

        @pl.when(m == pl.num_programs(0) - 1)
        def _():
            ent_ref[0] = ent_acc_ref[0]


def kernel(x, W1, b1, W2, b2):
    tokens, in_dim = x.shape
    hidden_dim, num_experts = W2.shape
    k = 8
    bm = 512
    bh = hidden_dim
    num_m = tokens // bm
    num_h = hidden_dim // bh

    b1r = b1.reshape(1, hidden_dim)
    b2r = b2.reshape(1, num_experts)

    body = functools.partial(_gating_body, num_h=num_h, k=k,
                             num_experts=num_experts)

    wts, idx, ent_sum = pl.pallas_call(
        body,
        grid=(num_m, num_h),
        in_specs=[
            pl.BlockSpec((bm, in_dim), lambda m, h: (m, 0)),
            pl.BlockSpec((in_dim, bh), lambda m, h: (0, h)),
            pl.BlockSpec((1, bh), lambda m, h: (0, h)),
            pl.BlockSpec((bh, num_experts), lambda m, h: (h, 0)),
            pl.BlockSpec((1, num_experts), lambda m, h: (0, 0)),
        ],
        out_specs=[
            pl.BlockSpec((bm, k), lambda m, h: (m, 0)),
            pl.BlockSpec((bm, k), lambda m, h: (m, 0)),
            pl.BlockSpec(memory_space=pltpu.SMEM),
        ],
        out_shape=[
            jax.ShapeDtypeStruct((tokens, k), jnp.float32),
            jax.ShapeDtypeStruct((tokens, k), jnp.int32),
            jax.ShapeDtypeStruct((1,), jnp.float32),
        ],
        scratch_shapes=[
            pltpu.VMEM((bm, num_experts), jnp.float32),
            pltpu.SMEM((1,), jnp.float32),
        ],
    )(x, W1, b1r, W2, b2r)

    uncertainty = (ent_sum[0] / tokens) / jnp.log(jnp.float32(num_experts))
    return wts, idx, uncertainty


# speedup vs baseline: 1.5316x; 1.1020x over previous
import functools

import jax
import jax.numpy as jnp
from jax.experimental import pallas as pl
from jax.experimental.pallas import tpu as pltpu


def _gating_body(x_ref, w1_ref, b1_ref, w2_ref, b2_ref,
                 wts_ref, idx_ref, ent_ref,
                 logits_ref, ent_acc_ref, *, k, num_experts, num_m):
    m = pl.program_id(0)

    logits = logits_ref[...] + b2_ref[...]
    mx = jnp.max(logits, axis=1, keepdims=True)
    e = jnp.exp(logits - mx)
    s = jnp.sum(e, axis=1, keepdims=True)
    probs = e / s

    ent = -jnp.sum(probs * jnp.log(probs + 1e-10), axis=1)
    ent_blk = jnp.sum(ent)

    bm = probs.shape[0]
    lane = jax.lax.broadcasted_iota(jnp.int32, (bm, num_experts), 1)
    pbits = jax.lax.bitcast_convert_type(probs, jnp.int32)
    key = jnp.bitwise_or(jnp.bitwise_and(pbits, jnp.int32(~63)),
                         (num_experts - 1) - lane)
    kms = []
    for _ in range(k):
        km = jnp.max(key, axis=1, keepdims=True)
        kms.append(km)
        key = jnp.where(key == km, jnp.int32(-1), key)
    km8 = jnp.concatenate(kms, axis=1)
    idx_ref[...] = (num_experts - 1) - jnp.bitwise_and(km8, jnp.int32(63))
    w = jax.lax.bitcast_convert_type(
        jnp.bitwise_and(km8, jnp.int32(~63)), jnp.float32)
    wts_ref[...] = w / jnp.sum(w, axis=1, keepdims=True)

    prev_acc = jnp.where(m == 0, 0.0, ent_acc_ref[0])
    new_acc = prev_acc + jnp.where(m == 0, 0.0, ent_blk)
    ent_acc_ref[0] = new_acc
    ent_ref[0] = new_acc

    hidden = jnp.dot(x_ref[...], w1_ref[...],
                     preferred_element_type=jnp.float32)
    hidden = jnp.maximum(hidden + b1_ref[...], 0.0)
    logits_ref[...] = jnp.dot(hidden, w2_ref[...],
                              preferred_element_type=jnp.float32)


def kernel(x, W1, b1, W2, b2):
    tokens, in_dim = x.shape
    hidden_dim, num_experts = W2.shape
    k = 8
    bm = 512
    num_m = tokens // bm

    b1r = b1.reshape(1, hidden_dim)
    b2r = b2.reshape(1, num_experts)

    body = functools.partial(_gating_body, k=k, num_experts=num_experts,
                             num_m=num_m)

    last = num_m - 1
    wts, idx, ent_sum = pl.pallas_call(
        body,
        grid=(num_m + 1,),
        in_specs=[
            pl.BlockSpec((bm, in_dim), lambda m: (min_clamp(m, last), 0)),
            pl.BlockSpec((in_dim, hidden_dim), lambda m: (0, 0)),
            pl.BlockSpec((1, hidden_dim), lambda m: (0, 0)),
            pl.BlockSpec((hidden_dim, num_experts), lambda m: (0, 0)),
            pl.BlockSpec((1, num_experts), lambda m: (0, 0)),
        ],
        out_specs=[
            pl.BlockSpec((bm, k), lambda m: (max_clamp(m - 1), 0)),
            pl.BlockSpec((bm, k), lambda m: (max_clamp(m - 1), 0)),
            pl.BlockSpec(memory_space=pltpu.SMEM),
        ],
        out_shape=[
            jax.ShapeDtypeStruct((tokens, k), jnp.float32),
            jax.ShapeDtypeStruct((tokens, k), jnp.int32),
            jax.ShapeDtypeStruct((1,), jnp.float32),
        ],
        scratch_shapes=[
            pltpu.VMEM((bm, num_experts), jnp.float32),
            pltpu.SMEM((1,), jnp.float32),
        ],
    )(x, W1, b1r, W2, b2r)

    uncertainty = (ent_sum[0] / tokens) / jnp.log(jnp.float32(num_experts))
    return wts, idx, uncertainty


def min_clamp(m, hi):
    return jnp.minimum(m, hi)


def max_clamp(m):
    return jnp.maximum(m, 0)


# speedup vs baseline: 1.5333x; 1.0011x over previous
import functools

import jax
import jax.numpy as jnp
from jax.experimental import pallas as pl
from jax.experimental.pallas import tpu as pltpu


def _epilogue(logits, b2, k, num_experts):
    logits = logits + b2
    mx = jnp.max(logits, axis=1, keepdims=True)
    e = jnp.exp(logits - mx)
    s = jnp.sum(e, axis=1, keepdims=True)
    probs = e / s

    ent_blk = jnp.sum(-jnp.sum(probs * jnp.log(probs + 1e-10), axis=1))

    bm = probs.shape[0]
    lane = jax.lax.broadcasted_iota(jnp.int32, (bm, num_experts), 1)
    pbits = jax.lax.bitcast_convert_type(probs, jnp.int32)
    key = jnp.bitwise_or(jnp.bitwise_and(pbits, jnp.int32(~63)),
                         (num_experts - 1) - lane)
    kms = []
    for _ in range(k):
        km = jnp.max(key, axis=1, keepdims=True)
        kms.append(km)
        key = jnp.where(key == km, jnp.int32(-1), key)
    km8 = jnp.concatenate(kms, axis=1)
    idx = (num_experts - 1) - jnp.bitwise_and(km8, jnp.int32(63))
    w = jax.lax.bitcast_convert_type(
        jnp.bitwise_and(km8, jnp.int32(~63)), jnp.float32)
    wts = w / jnp.sum(w, axis=1, keepdims=True)
    return wts, idx, ent_blk


def _main_body(x_ref, w1_ref, b1_ref, w2_ref, b2_ref,
               wts_ref, idx_ref, ent_ref, logits_ref,
               ent_acc_ref, *, k, num_experts):
    m = pl.program_id(0)

    wts, idx, ent_blk = _epilogue(logits_ref[...], b2_ref[...],
                                  k, num_experts)
    wts_ref[...] = wts
    idx_ref[...] = idx

    prev_acc = jnp.where(m == 0, 0.0, ent_acc_ref[0])
    new_acc = prev_acc + jnp.where(m == 0, 0.0, ent_blk)
    ent_acc_ref[0] = new_acc
    ent_ref[0] = new_acc

    hidden = jnp.dot(x_ref[...], w1_ref[...],
                     preferred_element_type=jnp.float32)
    hidden = jnp.maximum(hidden + b1_ref[...], 0.0)
    logits_ref[...] = jnp.dot(hidden, w2_ref[...],
                              preferred_element_type=jnp.float32)


def _last_body(logits_ref, b2_ref, ent_in_ref,
               wts_ref, idx_ref, ent_ref, *, k, num_experts):
    wts, idx, ent_blk = _epilogue(logits_ref[...], b2_ref[...],
                                  k, num_experts)
    wts_ref[...] = wts
    idx_ref[...] = idx
    ent_ref[0] = ent_in_ref[0] + ent_blk


def kernel(x, W1, b1, W2, b2):
    tokens, in_dim = x.shape
    hidden_dim, num_experts = W2.shape
    k = 8
    bm = 512
    num_m = tokens // bm

    b1r = b1.reshape(1, hidden_dim)
    b2r = b2.reshape(1, num_experts)

    main = functools.partial(_main_body, k=k, num_experts=num_experts)
    lastb = functools.partial(_last_body, k=k, num_experts=num_experts)

    wts_head, idx_head, ent_part, logits_last = pl.pallas_call(
        main,
        grid=(num_m,),
        in_specs=[
            pl.BlockSpec((bm, in_dim), lambda m: (m, 0)),
            pl.BlockSpec((in_dim, hidden_dim), lambda m: (0, 0)),
            pl.BlockSpec((1, hidden_dim), lambda m: (0, 0)),
            pl.BlockSpec((hidden_dim, num_experts), lambda m: (0, 0)),
            pl.BlockSpec((1, num_experts), lambda m: (0, 0)),
        ],
        out_specs=[
            pl.BlockSpec((bm, k), lambda m: (jnp.maximum(m - 1, 0), 0)),
            pl.BlockSpec((bm, k), lambda m: (jnp.maximum(m - 1, 0), 0)),
            pl.BlockSpec(memory_space=pltpu.SMEM),
            pl.BlockSpec((bm, num_experts), lambda m: (0, 0)),
        ],
        out_shape=[
            jax.ShapeDtypeStruct((tokens - bm, k), jnp.float32),
            jax.ShapeDtypeStruct((tokens - bm, k), jnp.int32),
            jax.ShapeDtypeStruct((1,), jnp.float32),
            jax.ShapeDtypeStruct((bm, num_experts), jnp.float32),
        ],
        scratch_shapes=[
            pltpu.SMEM((1,), jnp.float32),
        ],
    )(x, W1, b1r, W2, b2r)

    wts_tail, idx_tail, ent_sum = pl.pallas_call(
        lastb,
        in_specs=[
            pl.BlockSpec((bm, num_experts), lambda: (0, 0)),
            pl.BlockSpec((1, num_experts), lambda: (0, 0)),
            pl.BlockSpec(memory_space=pltpu.SMEM),
        ],
        out_specs=[
            pl.BlockSpec((bm, k), lambda: (0, 0)),
            pl.BlockSpec((bm, k), lambda: (0, 0)),
            pl.BlockSpec(memory_space=pltpu.SMEM),
        ],
        out_shape=[
            jax.ShapeDtypeStruct((bm, k), jnp.float32),
            jax.ShapeDtypeStruct((bm, k), jnp.int32),
            jax.ShapeDtypeStruct((1,), jnp.float32),
        ],
    )(logits_last, b2r, ent_part)

    wts = jnp.concatenate([wts_head, wts_tail], axis=0)
    idx = jnp.concatenate([idx_head, idx_tail], axis=0)
    uncertainty = (ent_sum[0] / tokens) / jnp.log(jnp.float32(num_experts))
    return wts, idx, uncertainty
